# Initial kernel scaffold; baseline (speedup 1.0000x reference)
import jax
import jax.numpy as jnp
from jax.experimental import pallas as pl


def kernel(V, counts):
    raise NotImplementedError("write your pallas kernel here")


# speedup vs baseline: 7.5214x; 7.5214x over previous
import jax
import jax.numpy as jnp
from jax import lax
from jax.experimental import pallas as pl
from jax.experimental.pallas import tpu as pltpu
from jax.experimental.pallas import tpu_sc as plsc

N = 5000
D = 32
BATCH = 64
THRESHOLD = 0.5
NPAD = 5120
NELEM = 5008
ROWBLK = 128
GRID = NPAD // ROWBLK
NW = NELEM // 16
NWROW = NPAD // 16


def _tc_body(v_blk, vt_all, counts_ref, s_out, scores_out):
    cos_thr = 2.0 * THRESHOLD - 1.0
    S = lax.dot_general(v_blk[...], vt_all[...], (((1,), (0,)), ((), ())),
                        preferred_element_type=jnp.float32)
    s_out[...] = S
    w = jnp.log1p(counts_ref[...].astype(jnp.float32))
    contrib = jnp.where(S >= cos_thr, S, 0.0) * w
    scores_out[...] = jnp.sum(contrib, axis=1).reshape(1, 1, ROWBLK)


def _sc_group_body(s_hbm, out_hbm, gids_v, mark_v, rowa, rowb, sema, semb):
    cid = lax.axis_index("c")
    sid = lax.axis_index("s")
    cos_thr = 2.0 * THRESHOLD - 1.0

    @pl.when(jnp.logical_and(cid == 0, sid == 0))
    def _():
        iota = lax.iota(jnp.int32, 16)

        def init_g(w, c):
            base = w * 16
            gids_v[pl.ds(base, 16)] = base + iota
            return c
        lax.fori_loop(0, NW, init_g, 0)

        def init_m(w, c):
            mark_v[pl.ds(w * 16, 16)] = jnp.full((16,), -1, jnp.int32)
            return c
        lax.fori_loop(0, NWROW, init_m, 0)

        pltpu.make_async_copy(s_hbm.at[0], rowa, sema).start()

        def process(r, buf):
            jlo = (r // BATCH) * BATCH + 1
            rv = jnp.full((16,), r, jnp.int32)
            gk = plsc.load_gather(gids_v, [rv])

            def pass_a(w, c):
                base = w * 16
                sv = buf[pl.ds(base, 16)]
                jv = base + iota
                m = (sv >= cos_thr) & (jv >= jlo) & (jv < N)
                g = gids_v[pl.ds(base, 16)]
                plsc.store_scatter(mark_v, [g], rv, mask=m)
                return c
            lax.fori_loop(0, NW, pass_a, 0)

            def pass_b(w, c):
                base = w * 16
                g = gids_v[pl.ds(base, 16)]
                t = plsc.load_gather(mark_v, [g])
                gids_v[pl.ds(base, 16)] = jnp.where(t == r, gk, g)
                return c
            lax.fori_loop(0, NW, pass_b, 0)

        def pair(p, c):
            r0 = 2 * p
            pltpu.make_async_copy(s_hbm.at[r0], rowa, sema).wait()
            pltpu.make_async_copy(s_hbm.at[r0 + 1], rowb, semb).start()
            process(r0, rowa)
            pltpu.make_async_copy(s_hbm.at[r0 + 1], rowb, semb).wait()
            pltpu.make_async_copy(s_hbm.at[r0 + 2], rowa, sema).start()
            process(r0 + 1, rowb)
            return c
        lax.fori_loop(0, N // 2, pair, 0)

        pltpu.make_async_copy(s_hbm.at[N], rowa, sema).wait()
        pltpu.sync_copy(gids_v, out_hbm)


def kernel(V, counts):
    v_pad = jnp.zeros((NPAD, D), jnp.float32).at[:N].set(V)
    vt_pad = v_pad.T
    counts_pad = jnp.zeros((1, NPAD), jnp.int32).at[0, :N].set(counts)

    S, scores2d = pl.pallas_call(
        _tc_body,
        grid=(GRID,),
        in_specs=[
            pl.BlockSpec((ROWBLK, D), lambda i: (i, 0)),
            pl.BlockSpec((D, NPAD), lambda i: (0, 0)),
            pl.BlockSpec((1, NPAD), lambda i: (0, 0)),
        ],
        out_specs=[
            pl.BlockSpec((ROWBLK, NPAD), lambda i: (i, 0)),
            pl.BlockSpec((1, 1, ROWBLK), lambda i: (i, 0, 0)),
        ],
        out_shape=[
            jax.ShapeDtypeStruct((NPAD, NPAD), jnp.float32),
            jax.ShapeDtypeStruct((GRID, 1, ROWBLK), jnp.float32),
        ],
    )(v_pad, vt_pad, counts_pad)

    gids = pl.kernel(
        _sc_group_body,
        out_type=jax.ShapeDtypeStruct((NELEM,), jnp.int32),
        mesh=plsc.VectorSubcoreMesh(core_axis_name="c", subcore_axis_name="s"),
        compiler_params=pltpu.CompilerParams(needs_layout_passes=False),
        scratch_types=[
            pltpu.VMEM((NELEM,), jnp.int32),
            pltpu.VMEM((NPAD,), jnp.int32),
            pltpu.VMEM((NPAD,), jnp.float32),
            pltpu.VMEM((NPAD,), jnp.float32),
            pltpu.SemaphoreType.DMA,
            pltpu.SemaphoreType.DMA,
        ],
    )(S)

    return (gids[:N], scores2d.reshape(NPAD)[:N])


# speedup vs baseline: 1273.0656x; 169.2584x over previous
import jax
import jax.numpy as jnp
from jax import lax
from jax.experimental import pallas as pl
from jax.experimental.pallas import tpu as pltpu
from jax.experimental.pallas import tpu_sc as plsc

N = 5000
D = 32
BATCH = 64
THRESHOLD = 0.5
NPAD = 5120
NELEM = 5008
ROWBLK = 128
GRID = NPAD // ROWBLK
NW = NELEM // 16
NWROW = NPAD // 16


def _tc_body(v_blk, vt_all, counts_ref, s_out, scores_out):
    cos_thr = 2.0 * THRESHOLD - 1.0
    S = lax.dot_general(v_blk[...], vt_all[...], (((1,), (0,)), ((), ())),
                        preferred_element_type=jnp.float32)
    s_out[...] = S
    w = jnp.log1p(counts_ref[...].astype(jnp.float32))
    contrib = jnp.where(S >= cos_thr, S, 0.0) * w
    scores_out[...] = jnp.sum(contrib, axis=1).reshape(1, 1, ROWBLK)


def _sc_group_body(s_hbm, out_hbm, gids_v, mark_v, row_v):
    cid = lax.axis_index("c")
    sid = lax.axis_index("s")
    cos_thr = 2.0 * THRESHOLD - 1.0

    @pl.when(jnp.logical_and(cid == 0, sid == 0))
    def _():
        iota = lax.iota(jnp.int32, 16)

        def init_g(w, c):
            base = w * 16
            g = base + iota
            gids_v[pl.ds(base, 16)] = jnp.where(g < N, g, 0)
            return c
        lax.fori_loop(0, NW, init_g, 0)

        def init_m(w, c):
            mark_v[pl.ds(w * 16, 16)] = jnp.full((16,), -1, jnp.int32)
            return c
        lax.fori_loop(0, NWROW, init_m, 0)

        def step(carry):
            r, _ = carry
            pltpu.sync_copy(s_hbm.at[r], row_v)
            jlo = (r // BATCH) * BATCH + 1
            rv = jnp.full((16,), r, jnp.int32)
            gk = plsc.load_gather(gids_v, [rv])

            def pass_a(w, c):
                base = w * 16
                sv = row_v[pl.ds(base, 16)]
                jv = base + iota
                m = (sv >= cos_thr) & (jv >= jlo) & (jv < N)
                g = gids_v[pl.ds(base, 16)]
                plsc.store_scatter(mark_v, [g], rv, mask=m)
                return c
            lax.fori_loop(0, NW, pass_a, 0)

            def pass_b(w, acc):
                base = w * 16
                g = gids_v[pl.ds(base, 16)]
                t = plsc.load_gather(mark_v, [g])
                ng = jnp.where(t == r, gk, g)
                gids_v[pl.ds(base, 16)] = ng
                return jnp.where(ng != gk, jnp.int32(1), acc)
            acc = lax.fori_loop(0, NW, pass_b, jnp.zeros((16,), jnp.int32))
            return r + 1, jnp.max(acc) == 0

        def cond(carry):
            r, done = carry
            return jnp.logical_and(r < N, jnp.logical_not(done))

        lax.while_loop(cond, step, (jnp.int32(0), jnp.bool_(False)))
        pltpu.sync_copy(gids_v, out_hbm)


def kernel(V, counts):
    v_pad = jnp.zeros((NPAD, D), jnp.float32).at[:N].set(V)
    vt_pad = v_pad.T
    counts_pad = jnp.zeros((1, NPAD), jnp.int32).at[0, :N].set(counts)

    S, scores2d = pl.pallas_call(
        _tc_body,
        grid=(GRID,),
        in_specs=[
            pl.BlockSpec((ROWBLK, D), lambda i: (i, 0)),
            pl.BlockSpec((D, NPAD), lambda i: (0, 0)),
            pl.BlockSpec((1, NPAD), lambda i: (0, 0)),
        ],
        out_specs=[
            pl.BlockSpec((ROWBLK, NPAD), lambda i: (i, 0)),
            pl.BlockSpec((1, 1, ROWBLK), lambda i: (i, 0, 0)),
        ],
        out_shape=[
            jax.ShapeDtypeStruct((NPAD, NPAD), jnp.float32),
            jax.ShapeDtypeStruct((GRID, 1, ROWBLK), jnp.float32),
        ],
    )(v_pad, vt_pad, counts_pad)

    gids = pl.kernel(
        _sc_group_body,
        out_type=jax.ShapeDtypeStruct((NELEM,), jnp.int32),
        mesh=plsc.VectorSubcoreMesh(core_axis_name="c", subcore_axis_name="s"),
        compiler_params=pltpu.CompilerParams(needs_layout_passes=False),
        scratch_types=[
            pltpu.VMEM((NELEM,), jnp.int32),
            pltpu.VMEM((NPAD,), jnp.int32),
            pltpu.VMEM((NPAD,), jnp.float32),
        ],
    )(S)

    return (gids[:N], scores2d.reshape(NPAD)[:N])


# speedup vs baseline: 1375.1037x; 1.0802x over previous
import jax
import jax.numpy as jnp
from jax import lax
from jax.experimental import pallas as pl
from jax.experimental.pallas import tpu as pltpu
from jax.experimental.pallas import tpu_sc as plsc

N = 5000
D = 32
BATCH = 64
THRESHOLD = 0.5
NPAD = 5120
NELEM = 5008
ROWBLK = 128
GRID = NPAD // ROWBLK
HEADBLKS = 2
HEAD = HEADBLKS * ROWBLK
NW = NELEM // 16
NWROW = NPAD // 16
COS_THR = 2.0 * THRESHOLD - 1.0


def _tc_scores_body(v_blk, vt_all, counts_ref, shead_out, scores_out):
    i = pl.program_id(0)
    S = lax.dot_general(v_blk[...], vt_all[...], (((1,), (0,)), ((), ())),
                        preferred_element_type=jnp.float32)
    w = jnp.log1p(counts_ref[...].astype(jnp.float32))
    contrib = jnp.where(S >= COS_THR, S, 0.0) * w
    scores_out[...] = jnp.sum(contrib, axis=1).reshape(1, 1, ROWBLK)

    @pl.when(i < HEADBLKS)
    def _():
        shead_out[...] = S


def _tc_full_s_body(v_blk, vt_all, s_out):
    s_out[...] = lax.dot_general(v_blk[...], vt_all[...],
                                 (((1,), (0,)), ((), ())),
                                 preferred_element_type=jnp.float32)


def _merge_steps(s_hbm, row_v, gids_v, mark_v, r_start, r_stop):
    iota = lax.iota(jnp.int32, 16)

    def step(carry):
        r, _ = carry
        pltpu.sync_copy(s_hbm.at[r], row_v)
        jlo = (r // BATCH) * BATCH + 1
        rv = jnp.full((16,), r, jnp.int32)
        gk = plsc.load_gather(gids_v, [rv])

        def pass_a(w, c):
            base = w * 16
            sv = row_v[pl.ds(base, 16)]
            jv = base + iota
            m = (sv >= COS_THR) & (jv >= jlo) & (jv < N)
            g = gids_v[pl.ds(base, 16)]
            plsc.store_scatter(mark_v, [g], rv, mask=m)
            return c
        lax.fori_loop(0, NW, pass_a, 0)

        def pass_b(w, acc):
            base = w * 16
            g = gids_v[pl.ds(base, 16)]
            t = plsc.load_gather(mark_v, [g])
            ng = jnp.where(t == r, gk, g)
            gids_v[pl.ds(base, 16)] = ng
            return jnp.where(ng != gk, jnp.int32(1), acc)
        acc = lax.fori_loop(0, NW, pass_b, jnp.zeros((16,), jnp.int32))
        return r + 1, jnp.max(acc) == 0

    def cond(carry):
        r, done = carry
        return jnp.logical_and(r < r_stop, jnp.logical_not(done))

    return lax.while_loop(cond, step, (r_start, jnp.bool_(False)))


def _sc_head_body(shead_hbm, gids_out, mark_out, state_out,
                  gids_v, mark_v, row_v, state_v):
    cid = lax.axis_index("c")
    sid = lax.axis_index("s")

    @pl.when(jnp.logical_and(cid == 0, sid == 0))
    def _():
        iota = lax.iota(jnp.int32, 16)

        def init_g(w, c):
            base = w * 16
            g = base + iota
            gids_v[pl.ds(base, 16)] = jnp.where(g < N, g, 0)
            return c
        lax.fori_loop(0, NW, init_g, 0)

        def init_m(w, c):
            mark_v[pl.ds(w * 16, 16)] = jnp.full((16,), -1, jnp.int32)
            return c
        lax.fori_loop(0, NWROW, init_m, 0)

        r_f, done_f = _merge_steps(shead_hbm, row_v, gids_v, mark_v,
                                   jnp.int32(0), jnp.int32(HEAD))

        done_i = jnp.where(done_f, jnp.int32(1), jnp.int32(0))
        state_v[...] = jnp.where(iota == 0, r_f,
                                 jnp.where(iota == 1, done_i, 0))
        pltpu.sync_copy(gids_v, gids_out)
        pltpu.sync_copy(mark_v, mark_out)
        pltpu.sync_copy(state_v, state_out)


def _sc_resume_body(s_hbm, gids_in, mark_in, state_in, gids_out,
                    gids_v, mark_v, row_v, state_v):
    cid = lax.axis_index("c")
    sid = lax.axis_index("s")

    @pl.when(jnp.logical_and(cid == 0, sid == 0))
    def _():
        iota = lax.iota(jnp.int32, 16)
        pltpu.sync_copy(gids_in, gids_v)
        pltpu.sync_copy(mark_in, mark_v)
        pltpu.sync_copy(state_in, state_v)
        sv = state_v[...]
        r0 = jnp.max(jnp.where(iota == 0, sv, 0))
        _merge_steps(s_hbm, row_v, gids_v, mark_v, r0, jnp.int32(N))
        pltpu.sync_copy(gids_v, gids_out)


_SC_MESH = plsc.VectorSubcoreMesh(core_axis_name="c", subcore_axis_name="s")
_SC_PARAMS = pltpu.CompilerParams(needs_layout_passes=False)


def kernel(V, counts):
    v_pad = jnp.zeros((NPAD, D), jnp.float32).at[:N].set(V)
    vt_pad = v_pad.T
    counts_pad = jnp.zeros((1, NPAD), jnp.int32).at[0, :N].set(counts)

    s_head, scores2d = pl.pallas_call(
        _tc_scores_body,
        grid=(GRID,),
        in_specs=[
            pl.BlockSpec((ROWBLK, D), lambda i: (i, 0)),
            pl.BlockSpec((D, NPAD), lambda i: (0, 0)),
            pl.BlockSpec((1, NPAD), lambda i: (0, 0)),
        ],
        out_specs=[
            pl.BlockSpec((ROWBLK, NPAD),
                         lambda i: (jnp.minimum(i, HEADBLKS - 1), 0)),
            pl.BlockSpec((1, 1, ROWBLK), lambda i: (i, 0, 0)),
        ],
        out_shape=[
            jax.ShapeDtypeStruct((HEAD, NPAD), jnp.float32),
            jax.ShapeDtypeStruct((GRID, 1, ROWBLK), jnp.float32),
        ],
    )(v_pad, vt_pad, counts_pad)

    gids_h, mark_h, state_h = pl.kernel(
        _sc_head_body,
        out_type=[
            jax.ShapeDtypeStruct((NELEM,), jnp.int32),
            jax.ShapeDtypeStruct((NPAD,), jnp.int32),
            jax.ShapeDtypeStruct((16,), jnp.int32),
        ],
        mesh=_SC_MESH,
        compiler_params=_SC_PARAMS,
        scratch_types=[
            pltpu.VMEM((NELEM,), jnp.int32),
            pltpu.VMEM((NPAD,), jnp.int32),
            pltpu.VMEM((NPAD,), jnp.float32),
            pltpu.VMEM((16,), jnp.int32),
        ],
    )(s_head)

    def finished(gids_h, mark_h, state_h):
        return gids_h[:N]

    def fallback(gids_h, mark_h, state_h):
        s_full = pl.pallas_call(
            _tc_full_s_body,
            grid=(GRID,),
            in_specs=[
                pl.BlockSpec((ROWBLK, D), lambda i: (i, 0)),
                pl.BlockSpec((D, NPAD), lambda i: (0, 0)),
            ],
            out_specs=pl.BlockSpec((ROWBLK, NPAD), lambda i: (i, 0)),
            out_shape=jax.ShapeDtypeStruct((NPAD, NPAD), jnp.float32),
        )(v_pad, vt_pad)
        gids_f = pl.kernel(
            _sc_resume_body,
            out_type=jax.ShapeDtypeStruct((NELEM,), jnp.int32),
            mesh=_SC_MESH,
            compiler_params=_SC_PARAMS,
            scratch_types=[
                pltpu.VMEM((NELEM,), jnp.int32),
                pltpu.VMEM((NPAD,), jnp.int32),
                pltpu.VMEM((NPAD,), jnp.float32),
                pltpu.VMEM((16,), jnp.int32),
            ],
        )(s_full, gids_h, mark_h, state_h)
        return gids_f[:N]

    gids = lax.cond(state_h[1] == 1, finished, fallback,
                    gids_h, mark_h, state_h)
    return (gids, scores2d.reshape(NPAD)[:N])


# speedup vs baseline: 2569.7437x; 1.8688x over previous
import jax
import jax.numpy as jnp
from jax import lax
from jax.experimental import pallas as pl
from jax.experimental.pallas import tpu as pltpu
from jax.experimental.pallas import tpu_sc as plsc

N = 5000
D = 32
BATCH = 64
THRESHOLD = 0.5
NPAD = 5120
NELEM = 5008
ROWBLK = 128
GRID = NPAD // ROWBLK
HEADBLKS = 2
HEAD = HEADBLKS * ROWBLK
NW = NELEM // 16
NWROW = NPAD // 16
COS_THR = 2.0 * THRESHOLD - 1.0


def _tc_scores_body(v_blk, vt_all, counts_ref, shead_out, scores_out):
    i = pl.program_id(0)
    S = lax.dot_general(v_blk[...], vt_all[...], (((1,), (0,)), ((), ())),
                        preferred_element_type=jnp.float32)
    w = jnp.log1p(counts_ref[...].astype(jnp.float32))
    contrib = jnp.where(S >= COS_THR, S, 0.0) * w
    scores_out[...] = jnp.sum(contrib, axis=1).reshape(1, 1, ROWBLK)

    @pl.when(i < HEADBLKS)
    def _():
        shead_out[...] = S


def _tc_full_s_body(v_blk, vt_all, s_out):
    s_out[...] = lax.dot_general(v_blk[...], vt_all[...],
                                 (((1,), (0,)), ((), ())),
                                 preferred_element_type=jnp.float32)


def _merge_steps(s_hbm, row_v, gids_v, mark_v, r_start, r_stop):
    iota = lax.iota(jnp.int32, 16)

    def step(carry):
        r, _ = carry
        pltpu.sync_copy(s_hbm.at[r], row_v)
        jlo = (r // BATCH) * BATCH + 1
        rv = jnp.full((16,), r, jnp.int32)
        gk = plsc.load_gather(gids_v, [rv])

        @plsc.parallel_loop(0, NELEM, step=16, unroll=8)
        def _pass_a(base):
            sv = row_v[pl.ds(base, 16)]
            jv = base + iota
            m = (sv >= COS_THR) & (jv >= jlo) & (jv < N)
            g = gids_v[pl.ds(base, 16)]
            plsc.store_scatter(mark_v, [g], rv, mask=m)

        @plsc.parallel_loop(0, NELEM, step=16, unroll=8,
                            carry=jnp.zeros((16,), jnp.int32))
        def acc(base, a):
            g = gids_v[pl.ds(base, 16)]
            t = plsc.load_gather(mark_v, [g])
            ng = jnp.where(t == r, gk, g)
            gids_v[pl.ds(base, 16)] = ng
            return jnp.where(ng != gk, jnp.int32(1), a)
        return r + 1, jnp.max(acc) == 0

    def cond(carry):
        r, done = carry
        return jnp.logical_and(r < r_stop, jnp.logical_not(done))

    return lax.while_loop(cond, step, (r_start, jnp.bool_(False)))


def _sc_head_body(shead_hbm, gids_out, mark_out, state_out,
                  gids_v, mark_v, row_v, state_v):
    cid = lax.axis_index("c")
    sid = lax.axis_index("s")

    @pl.when(jnp.logical_and(cid == 0, sid == 0))
    def _():
        iota = lax.iota(jnp.int32, 16)

        @plsc.parallel_loop(0, NELEM, step=16, unroll=8)
        def _init_g(base):
            g = base + iota
            gids_v[pl.ds(base, 16)] = jnp.where(g < N, g, 0)

        @plsc.parallel_loop(0, NPAD, step=16, unroll=8)
        def _init_m(base):
            mark_v[pl.ds(base, 16)] = jnp.full((16,), -1, jnp.int32)

        r_f, done_f = _merge_steps(shead_hbm, row_v, gids_v, mark_v,
                                   jnp.int32(0), jnp.int32(HEAD))

        done_i = jnp.where(done_f, jnp.int32(1), jnp.int32(0))
        state_v[...] = jnp.where(iota == 0, r_f,
                                 jnp.where(iota == 1, done_i, 0))
        pltpu.sync_copy(gids_v, gids_out)
        pltpu.sync_copy(mark_v, mark_out)
        pltpu.sync_copy(state_v, state_out)


def _sc_resume_body(s_hbm, gids_in, mark_in, state_in, gids_out,
                    gids_v, mark_v, row_v, state_v):
    cid = lax.axis_index("c")
    sid = lax.axis_index("s")

    @pl.when(jnp.logical_and(cid == 0, sid == 0))
    def _():
        iota = lax.iota(jnp.int32, 16)
        pltpu.sync_copy(gids_in, gids_v)
        pltpu.sync_copy(mark_in, mark_v)
        pltpu.sync_copy(state_in, state_v)
        sv = state_v[...]
        r0 = jnp.max(jnp.where(iota == 0, sv, 0))
        _merge_steps(s_hbm, row_v, gids_v, mark_v, r0, jnp.int32(N))
        pltpu.sync_copy(gids_v, gids_out)


_SC_MESH = plsc.VectorSubcoreMesh(core_axis_name="c", subcore_axis_name="s")
_SC_PARAMS = pltpu.CompilerParams(needs_layout_passes=False)


def kernel(V, counts):
    v_pad = jnp.zeros((NPAD, D), jnp.float32).at[:N].set(V)
    vt_pad = v_pad.T
    counts_pad = jnp.zeros((1, NPAD), jnp.int32).at[0, :N].set(counts)

    s_head, scores2d = pl.pallas_call(
        _tc_scores_body,
        grid=(GRID,),
        in_specs=[
            pl.BlockSpec((ROWBLK, D), lambda i: (i, 0)),
            pl.BlockSpec((D, NPAD), lambda i: (0, 0)),
            pl.BlockSpec((1, NPAD), lambda i: (0, 0)),
        ],
        out_specs=[
            pl.BlockSpec((ROWBLK, NPAD),
                         lambda i: (jnp.minimum(i, HEADBLKS - 1), 0)),
            pl.BlockSpec((1, 1, ROWBLK), lambda i: (i, 0, 0)),
        ],
        out_shape=[
            jax.ShapeDtypeStruct((HEAD, NPAD), jnp.float32),
            jax.ShapeDtypeStruct((GRID, 1, ROWBLK), jnp.float32),
        ],
    )(v_pad, vt_pad, counts_pad)

    gids_h, mark_h, state_h = pl.kernel(
        _sc_head_body,
        out_type=[
            jax.ShapeDtypeStruct((NELEM,), jnp.int32),
            jax.ShapeDtypeStruct((NPAD,), jnp.int32),
            jax.ShapeDtypeStruct((16,), jnp.int32),
        ],
        mesh=_SC_MESH,
        compiler_params=_SC_PARAMS,
        scratch_types=[
            pltpu.VMEM((NELEM,), jnp.int32),
            pltpu.VMEM((NPAD,), jnp.int32),
            pltpu.VMEM((NPAD,), jnp.float32),
            pltpu.VMEM((16,), jnp.int32),
        ],
    )(s_head)

    def finished(gids_h, mark_h, state_h):
        return gids_h[:N]

    def fallback(gids_h, mark_h, state_h):
        s_full = pl.pallas_call(
            _tc_full_s_body,
            grid=(GRID,),
            in_specs=[
                pl.BlockSpec((ROWBLK, D), lambda i: (i, 0)),
                pl.BlockSpec((D, NPAD), lambda i: (0, 0)),
            ],
            out_specs=pl.BlockSpec((ROWBLK, NPAD), lambda i: (i, 0)),
            out_shape=jax.ShapeDtypeStruct((NPAD, NPAD), jnp.float32),
        )(v_pad, vt_pad)
        gids_f = pl.kernel(
            _sc_resume_body,
            out_type=jax.ShapeDtypeStruct((NELEM,), jnp.int32),
            mesh=_SC_MESH,
            compiler_params=_SC_PARAMS,
            scratch_types=[
                pltpu.VMEM((NELEM,), jnp.int32),
                pltpu.VMEM((NPAD,), jnp.int32),
                pltpu.VMEM((NPAD,), jnp.float32),
                pltpu.VMEM((16,), jnp.int32),
            ],
        )(s_full, gids_h, mark_h, state_h)
        return gids_f[:N]

    gids = lax.cond(state_h[1] == 1, finished, fallback,
                    gids_h, mark_h, state_h)
    return (gids, scores2d.reshape(NPAD)[:N])
